# baseline (speedup 1.0000x reference)
import functools
import math

import jax
import jax.numpy as jnp
from jax import lax
from jax.experimental import pallas as pl
from jax.experimental.pallas import tpu as pltpu
from jax.experimental.pallas import tpu_sc as plsc

B, S, D, H = 1, 2048, 1024, 16
V, E, TOPK = 2, 8, 2
DFF, DFFG = 1024, 2048
DH = D // H

BT = 256
BQ = 256
G = 256
P = 6144
NB = P // G
NBT = V * NB
VS = V * S
NA = V * S * TOPK

_NC, _NS = 2, 16
NW = _NC * _NS
SLOTS = 2 * P // NW
TT = S // NW
SENT = S


def _qkv_body(x_ref, w_ref, b_ref, o_ref):
    o_ref[...] = (
        jnp.dot(x_ref[...], w_ref[...], preferred_element_type=jnp.float32)
        + b_ref[...]
    )


def _attn_body(q_ref, k_ref, v_ref, o_ref):
    si = pl.program_id(1)
    q = q_ref[0, :, :]
    k = k_ref[0, :, :]
    s = lax.dot_general(
        q, k, (((1,), (1,)), ((), ())), preferred_element_type=jnp.float32
    ) / math.sqrt(DH)
    rows = si * BQ + lax.broadcasted_iota(jnp.int32, (BQ, S), 0)
    cols = lax.broadcasted_iota(jnp.int32, (BQ, S), 1)
    s = jnp.where(cols > rows, jnp.float32(-1e9), s)
    p = jax.nn.softmax(s, axis=-1)
    o_ref[0, :, :] = jnp.dot(p, v_ref[0, :, :], preferred_element_type=jnp.float32)


def _oproj_ln_body(o_ref, w_ref, b_ref, x_ref, g_ref, beta_ref, out_ref):
    y = (
        jnp.dot(o_ref[...], w_ref[...], preferred_element_type=jnp.float32)
        + b_ref[...]
        + x_ref[...]
    )
    m = jnp.mean(y, axis=-1, keepdims=True)
    v = jnp.mean((y - m) ** 2, axis=-1, keepdims=True)
    out_ref[...] = (y - m) * lax.rsqrt(v + 1e-5) * g_ref[...] + beta_ref[...]


def _cumsum_rows(x):
    n = x.shape[0]
    s = 1
    while s < n:
        x = x + jnp.concatenate(
            [jnp.zeros((s, x.shape[1]), x.dtype), x[:-s, :]], axis=0
        )
        s *= 2
    return x


def _route_body(lg_ref, mk_ref, posk_ref, gatek_ref, gbe_ref, guide_ref):
    lg = lg_ref[...]
    probs = jax.nn.softmax(lg, axis=-1)
    iota_e = lax.broadcasted_iota(jnp.int32, (VS, E), 1)
    m1 = jnp.max(probs, axis=-1, keepdims=True)
    i1 = jnp.min(jnp.where(probs == m1, iota_e, E), axis=-1, keepdims=True)
    oh1 = iota_e == i1
    p2 = jnp.where(oh1, jnp.float32(-1.0), probs)
    m2 = jnp.max(p2, axis=-1, keepdims=True)
    i2 = jnp.min(jnp.where(p2 == m2, iota_e, E), axis=-1, keepdims=True)
    oh2 = iota_e == i2
    ssum = m1 + m2
    gatek_ref[...] = jnp.concatenate([m1 / ssum, m2 / ssum], axis=1)

    mk = mk_ref[...]
    mn = mk / (jnp.sum(mk, axis=-1, keepdims=True) + 1e-9)
    guide_ref[...] = (-jnp.sum(mn * jnp.log(probs + 1e-9)) / (S * V)).reshape(1, 1)

    cnt = (oh1 | oh2).astype(jnp.int32)
    C = _cumsum_rows(cnt)
    n0 = C[S - 1:S, :]
    n1 = C[VS - 1:VS, :] - n0
    rows = lax.broadcasted_iota(jnp.int32, (VS, 1), 0)
    is_v1 = rows >= S
    excl = C - cnt - jnp.where(is_v1, 1, 0) * n0
    np0 = ((n0 + (G - 1)) // G) * G
    np1 = ((n1 + (G - 1)) // G) * G
    tri = (
        lax.broadcasted_iota(jnp.int32, (E, E), 0)
        < lax.broadcasted_iota(jnp.int32, (E, E), 1)
    ).astype(jnp.float32)
    po0 = jnp.dot(
        np0.astype(jnp.float32), tri, preferred_element_type=jnp.float32
    ).astype(jnp.int32)
    po1 = jnp.dot(
        np1.astype(jnp.float32), tri, preferred_element_type=jnp.float32
    ).astype(jnp.int32)
    po_full = jnp.where(is_v1, po1, po0)
    base = po_full + excl + jnp.where(is_v1, P, 0)
    pos0 = jnp.sum(jnp.where(oh1, base, 0), axis=-1, keepdims=True)
    pos1 = jnp.sum(jnp.where(oh2, base, 0), axis=-1, keepdims=True)
    posk_ref[...] = jnp.concatenate([pos0, pos1], axis=1)

    r = lax.broadcasted_iota(jnp.int32, (NBT, 1), 0)
    isb1 = r >= NB
    iloc = jnp.where(isb1, r - NB, r)
    ends = jnp.where(isb1, po1 + np1, po0 + np0)
    cntb = jnp.sum((ends <= iloc * G).astype(jnp.int32), axis=-1, keepdims=True)
    gbe_ref[...] = jnp.minimum(cntb, E - 1) + jnp.where(isb1, E, 0)


def _gmm_body(gbe_ref, x_ref, w1_ref, b1_ref, w2_ref, b2_ref, eo_ref):
    x = x_ref[...]
    h = jax.nn.gelu(
        jnp.dot(x, w1_ref[0, :, :], preferred_element_type=jnp.float32)
        + b1_ref[0, :, :]
    )
    eo_ref[...] = (
        jnp.dot(h, w2_ref[0, :, :], preferred_element_type=jnp.float32)
        + b2_ref[0, :, :]
    )


def _final_body(
    x1_ref, g4_ref, gk_ref, w1_ref, b1_ref, w2_ref, b2_ref, g_ref, beta_ref, out_ref
):
    x = x1_ref[...]
    h = jax.nn.gelu(
        jnp.dot(x, w1_ref[...], preferred_element_type=jnp.float32) + b1_ref[...]
    )
    fin = (
        jnp.dot(h, w2_ref[...], preferred_element_type=jnp.float32)
        + b2_ref[...]
        + x
    )
    gk = gk_ref[...]
    for j in range(V * TOPK):
        fin = fin + g4_ref[j, :, :] * gk[:, j:j + 1]
    m = jnp.mean(fin, axis=-1, keepdims=True)
    v = jnp.mean((fin - m) ** 2, axis=-1, keepdims=True)
    out_ref[...] = (fin - m) * lax.rsqrt(v + 1e-5) * g_ref[...] + beta_ref[...]


_GCH = 64


@functools.lru_cache(maxsize=None)
def _sc_kernels():
    mesh = plsc.VectorSubcoreMesh(core_axis_name="c", subcore_axis_name="s")

    @functools.partial(
        pl.kernel,
        mesh=mesh,
        compiler_params=pltpu.CompilerParams(needs_layout_passes=False),
        out_type=jax.ShapeDtypeStruct((2 * P,), jnp.int32),
        scratch_types=[
            pltpu.VMEM((NA,), jnp.int32),
            pltpu.VMEM((SLOTS,), jnp.int32),
        ],
    )
    def sc_scatter(pos_hbm, tok_hbm, posv, tokb):
        wid = lax.axis_index("s") * _NC + lax.axis_index("c")
        base = wid * SLOTS
        pltpu.sync_copy(pos_hbm, posv)

        def init(i, carry):
            tokb[pl.ds(i * 16, 16)] = jnp.full((16,), SENT, jnp.int32)
            return carry

        lax.fori_loop(0, SLOTS // 16, init, 0)
        iota16 = lax.broadcasted_iota(jnp.int32, (16,), 0)

        def body(c, carry):
            pv = posv[pl.ds(c * 16, 16)]
            av = c * 16 + iota16
            tv = jnp.bitwise_and(av, S - 1)
            rel = pv - base
            msk = (rel >= 0) & (rel < SLOTS)
            relc = jnp.clip(rel, 0, SLOTS - 1)
            plsc.store_scatter(tokb, [relc], tv, mask=msk)
            return carry

        lax.fori_loop(0, NA // 16, body, 0)
        pltpu.sync_copy(tokb, tok_hbm.at[pl.ds(base, SLOTS)])

    @functools.partial(
        pl.kernel,
        mesh=mesh,
        compiler_params=pltpu.CompilerParams(needs_layout_passes=False),
        out_type=jax.ShapeDtypeStruct((2 * P, D), jnp.float32),
        scratch_types=[
            pltpu.VMEM((SLOTS,), jnp.int32),
            pltpu.VMEM((_GCH, D), jnp.float32),
            pltpu.SemaphoreType.DMA,
        ],
    )
    def sc_gather_x(tok_hbm, xpad_hbm, xs_hbm, tokv, rows, sem):
        wid = lax.axis_index("s") * _NC + lax.axis_index("c")
        base = wid * SLOTS
        pltpu.sync_copy(tok_hbm.at[pl.ds(base, SLOTS)], tokv)

        def body(ch, carry):
            idx = tokv.at[pl.ds(ch * _GCH, _GCH)]
            pltpu.async_copy(xpad_hbm.at[idx], rows, sem).wait()
            pltpu.sync_copy(rows, xs_hbm.at[pl.ds(base + ch * _GCH, _GCH)])
            return carry

        lax.fori_loop(0, SLOTS // _GCH, body, 0)

    @functools.partial(
        pl.kernel,
        mesh=mesh,
        compiler_params=pltpu.CompilerParams(needs_layout_passes=False),
        out_type=jax.ShapeDtypeStruct((V * TOPK * S, D), jnp.float32),
        scratch_types=[
            pltpu.VMEM((TT,), jnp.int32),
            pltpu.VMEM((TT, D), jnp.float32),
            pltpu.SemaphoreType.DMA,
        ],
    )
    def sc_gather_eo(pos_hbm, eo_hbm, g4_hbm, pv, rows, sem):
        wid = lax.axis_index("s") * _NC + lax.axis_index("c")
        tbase = wid * TT
        for j in range(V * TOPK):
            pltpu.sync_copy(pos_hbm.at[pl.ds(j * S + tbase, TT)], pv)
            pltpu.async_copy(eo_hbm.at[pv], rows, sem).wait()
            pltpu.sync_copy(rows, g4_hbm.at[pl.ds(j * S + tbase, TT)])

    return sc_scatter, sc_gather_x, sc_gather_eo


def kernel(x, total_logits, total_masks, attn_mask, Wq, bq, Wk, bk, Wv, bv, Wo, bo,
           g1, beta1, g2, beta2, W1v, b1v, W2v, b2v, W1g, b1g, W2g, b2g):
    f32 = jnp.float32
    xf = x.reshape(S, D)

    lg = total_logits.reshape(VS, E)
    mk = total_masks.reshape(VS, E)
    posk, gatek, gbe2, guide2 = pl.pallas_call(
        _route_body,
        in_specs=[
            pl.BlockSpec((VS, E), lambda: (0, 0)),
            pl.BlockSpec((VS, E), lambda: (0, 0)),
        ],
        out_specs=[
            pl.BlockSpec((VS, TOPK), lambda: (0, 0)),
            pl.BlockSpec((VS, TOPK), lambda: (0, 0)),
            pl.BlockSpec((NBT, 1), lambda: (0, 0)),
            pl.BlockSpec((1, 1), lambda: (0, 0)),
        ],
        out_shape=[
            jax.ShapeDtypeStruct((VS, TOPK), jnp.int32),
            jax.ShapeDtypeStruct((VS, TOPK), f32),
            jax.ShapeDtypeStruct((NBT, 1), jnp.int32),
            jax.ShapeDtypeStruct((1, 1), f32),
        ],
    )(lg, mk)
    total_guide = guide2[0, 0]
    pos4 = posk.reshape(V, S, TOPK).transpose(0, 2, 1).reshape(NA)
    gate4 = gatek.reshape(V, S, TOPK).transpose(1, 0, 2).reshape(S, V * TOPK)
    gbe = gbe2.reshape(NBT)

    sc_scatter, sc_gather_x, sc_gather_eo = _sc_kernels()
    tokbuf = sc_scatter(pos4)

    Wqkv = jnp.concatenate([Wq, Wk, Wv], axis=1)
    bqkv = jnp.concatenate([bq, bk, bv]).reshape(1, 3 * D)
    qkv = pl.pallas_call(
        _qkv_body,
        grid=(S // BT,),
        in_specs=[
            pl.BlockSpec((BT, D), lambda i: (i, 0)),
            pl.BlockSpec((D, 3 * D), lambda i: (0, 0)),
            pl.BlockSpec((1, 3 * D), lambda i: (0, 0)),
        ],
        out_specs=pl.BlockSpec((BT, 3 * D), lambda i: (i, 0)),
        out_shape=jax.ShapeDtypeStruct((S, 3 * D), f32),
    )(xf, Wqkv, bqkv)

    q = qkv[:, :D].reshape(S, H, DH).transpose(1, 0, 2)
    k = qkv[:, D:2 * D].reshape(S, H, DH).transpose(1, 0, 2)
    v = qkv[:, 2 * D:].reshape(S, H, DH).transpose(1, 0, 2)

    o = pl.pallas_call(
        _attn_body,
        grid=(H, S // BQ),
        in_specs=[
            pl.BlockSpec((1, BQ, DH), lambda h, i: (h, i, 0)),
            pl.BlockSpec((1, S, DH), lambda h, i: (h, 0, 0)),
            pl.BlockSpec((1, S, DH), lambda h, i: (h, 0, 0)),
        ],
        out_specs=pl.BlockSpec((1, BQ, DH), lambda h, i: (h, i, 0)),
        out_shape=jax.ShapeDtypeStruct((H, S, DH), f32),
    )(q, k, v)
    o2 = o.transpose(1, 0, 2).reshape(S, D)

    x1 = pl.pallas_call(
        _oproj_ln_body,
        grid=(S // BT,),
        in_specs=[
            pl.BlockSpec((BT, D), lambda i: (i, 0)),
            pl.BlockSpec((D, D), lambda i: (0, 0)),
            pl.BlockSpec((1, D), lambda i: (0, 0)),
            pl.BlockSpec((BT, D), lambda i: (i, 0)),
            pl.BlockSpec((1, D), lambda i: (0, 0)),
            pl.BlockSpec((1, D), lambda i: (0, 0)),
        ],
        out_specs=pl.BlockSpec((BT, D), lambda i: (i, 0)),
        out_shape=jax.ShapeDtypeStruct((S, D), f32),
    )(o2, Wo, bo.reshape(1, D), xf, g1.reshape(1, D), beta1.reshape(1, D))

    xpad = jnp.concatenate([x1, jnp.zeros((8, D), f32)], axis=0)
    xs = sc_gather_x(tokbuf, xpad)

    W1r = W1v.reshape(V * E, D, DFF)
    b1r = b1v.reshape(V * E, 1, DFF)
    W2r = W2v.reshape(V * E, DFF, D)
    b2r = b2v.reshape(V * E, 1, D)
    eo = pl.pallas_call(
        _gmm_body,
        grid_spec=pltpu.PrefetchScalarGridSpec(
            num_scalar_prefetch=1,
            grid=(NBT,),
            in_specs=[
                pl.BlockSpec((G, D), lambda i, gbe_r: (i, 0)),
                pl.BlockSpec((1, D, DFF), lambda i, gbe_r: (gbe_r[i], 0, 0)),
                pl.BlockSpec((1, 1, DFF), lambda i, gbe_r: (gbe_r[i], 0, 0)),
                pl.BlockSpec((1, DFF, D), lambda i, gbe_r: (gbe_r[i], 0, 0)),
                pl.BlockSpec((1, 1, D), lambda i, gbe_r: (gbe_r[i], 0, 0)),
            ],
            out_specs=pl.BlockSpec((G, D), lambda i, gbe_r: (i, 0)),
        ),
        out_shape=jax.ShapeDtypeStruct((2 * P, D), f32),
    )(gbe, xs, W1r, b1r, W2r, b2r)

    g4 = sc_gather_eo(pos4, eo).reshape(V * TOPK, S, D)

    out = pl.pallas_call(
        _final_body,
        grid=(S // BT,),
        in_specs=[
            pl.BlockSpec((BT, D), lambda i: (i, 0)),
            pl.BlockSpec((V * TOPK, BT, D), lambda i: (0, i, 0)),
            pl.BlockSpec((BT, V * TOPK), lambda i: (i, 0)),
            pl.BlockSpec((D, DFFG), lambda i: (0, 0)),
            pl.BlockSpec((1, DFFG), lambda i: (0, 0)),
            pl.BlockSpec((DFFG, D), lambda i: (0, 0)),
            pl.BlockSpec((1, D), lambda i: (0, 0)),
            pl.BlockSpec((1, D), lambda i: (0, 0)),
            pl.BlockSpec((1, D), lambda i: (0, 0)),
        ],
        out_specs=pl.BlockSpec((BT, D), lambda i: (i, 0)),
        out_shape=jax.ShapeDtypeStruct((S, D), f32),
    )(
        x1, g4, gate4, W1g, b1g.reshape(1, DFFG), W2g, b2g.reshape(1, D),
        g2.reshape(1, D), beta2.reshape(1, D),
    )

    return out.reshape(B, S, D), total_guide


# speedup vs baseline: 1.0811x; 1.0811x over previous
import functools
import math

import jax
import jax.numpy as jnp
from jax import lax
from jax.experimental import pallas as pl
from jax.experimental.pallas import tpu as pltpu
from jax.experimental.pallas import tpu_sc as plsc

B, S, D, H = 1, 2048, 1024, 16
V, E, TOPK = 2, 8, 2
DFF, DFFG = 1024, 2048
DH = D // H

BT = 256
BQ = 256
G = 256
P = 6144
NB = P // G
NBT = V * NB
VS = V * S
NA = V * S * TOPK

_NC, _NS = 2, 16
NW = _NC * _NS
SLOTS = 2 * P // NW
TT = S // NW
SENT = S


def _qkv_body(x_ref, w_ref, b_ref, o_ref):
    acc = jnp.dot(
        x_ref[...].astype(jnp.bfloat16),
        w_ref[...].astype(jnp.bfloat16),
        preferred_element_type=jnp.float32,
    )
    o_ref[...] = (acc + b_ref[...]).astype(jnp.bfloat16)


def _attn_body(q_ref, k_ref, v_ref, o_ref):
    si = pl.program_id(1)
    q = q_ref[0, :, :]
    k = k_ref[0, :, :]
    s = lax.dot_general(
        q, k, (((1,), (1,)), ((), ())), preferred_element_type=jnp.float32
    ) / math.sqrt(DH)
    rows = si * BQ + lax.broadcasted_iota(jnp.int32, (BQ, S), 0)
    cols = lax.broadcasted_iota(jnp.int32, (BQ, S), 1)
    s = jnp.where(cols > rows, jnp.float32(-1e9), s)
    p = jax.nn.softmax(s, axis=-1)
    o_ref[0, :, :] = jnp.dot(
        p.astype(jnp.bfloat16), v_ref[0, :, :], preferred_element_type=jnp.float32
    ).astype(jnp.bfloat16)


def _oproj_ln_body(o_ref, w_ref, b_ref, x_ref, g_ref, beta_ref, out_ref):
    y = (
        jnp.dot(
            o_ref[...],
            w_ref[...].astype(jnp.bfloat16),
            preferred_element_type=jnp.float32,
        )
        + b_ref[...]
        + x_ref[...]
    )
    m = jnp.mean(y, axis=-1, keepdims=True)
    v = jnp.mean((y - m) ** 2, axis=-1, keepdims=True)
    out_ref[...] = (y - m) * lax.rsqrt(v + 1e-5) * g_ref[...] + beta_ref[...]


def _cumsum_rows(x):
    n = x.shape[0]
    s = 1
    while s < n:
        x = x + jnp.concatenate(
            [jnp.zeros((s, x.shape[1]), x.dtype), x[:-s, :]], axis=0
        )
        s *= 2
    return x


def _route_body(lg_ref, mk_ref, posk_ref, gatek_ref, gbe_ref, guide_ref):
    lg = lg_ref[...]
    probs = jax.nn.softmax(lg, axis=-1)
    iota_e = lax.broadcasted_iota(jnp.int32, (VS, E), 1)
    m1 = jnp.max(probs, axis=-1, keepdims=True)
    i1 = jnp.min(jnp.where(probs == m1, iota_e, E), axis=-1, keepdims=True)
    oh1 = iota_e == i1
    p2 = jnp.where(oh1, jnp.float32(-1.0), probs)
    m2 = jnp.max(p2, axis=-1, keepdims=True)
    i2 = jnp.min(jnp.where(p2 == m2, iota_e, E), axis=-1, keepdims=True)
    oh2 = iota_e == i2
    ssum = m1 + m2
    gatek_ref[...] = jnp.concatenate([m1 / ssum, m2 / ssum], axis=1)

    mk = mk_ref[...]
    mn = mk / (jnp.sum(mk, axis=-1, keepdims=True) + 1e-9)
    guide_ref[...] = (-jnp.sum(mn * jnp.log(probs + 1e-9)) / (S * V)).reshape(1, 1)

    cnt = (oh1 | oh2).astype(jnp.int32)
    C = _cumsum_rows(cnt)
    n0 = C[S - 1:S, :]
    n1 = C[VS - 1:VS, :] - n0
    rows = lax.broadcasted_iota(jnp.int32, (VS, 1), 0)
    is_v1 = rows >= S
    excl = C - cnt - jnp.where(is_v1, 1, 0) * n0
    np0 = ((n0 + (G - 1)) // G) * G
    np1 = ((n1 + (G - 1)) // G) * G
    tri = (
        lax.broadcasted_iota(jnp.int32, (E, E), 0)
        < lax.broadcasted_iota(jnp.int32, (E, E), 1)
    ).astype(jnp.float32)
    po0 = jnp.dot(
        np0.astype(jnp.float32), tri, preferred_element_type=jnp.float32
    ).astype(jnp.int32)
    po1 = jnp.dot(
        np1.astype(jnp.float32), tri, preferred_element_type=jnp.float32
    ).astype(jnp.int32)
    po_full = jnp.where(is_v1, po1, po0)
    base = po_full + excl + jnp.where(is_v1, P, 0)
    pos0 = jnp.sum(jnp.where(oh1, base, 0), axis=-1, keepdims=True)
    pos1 = jnp.sum(jnp.where(oh2, base, 0), axis=-1, keepdims=True)
    posk_ref[...] = jnp.concatenate([pos0, pos1], axis=1)

    r = lax.broadcasted_iota(jnp.int32, (NBT, 1), 0)
    isb1 = r >= NB
    iloc = jnp.where(isb1, r - NB, r)
    ends = jnp.where(isb1, po1 + np1, po0 + np0)
    cntb = jnp.sum((ends <= iloc * G).astype(jnp.int32), axis=-1, keepdims=True)
    gbe_ref[...] = jnp.minimum(cntb, E - 1) + jnp.where(isb1, E, 0)


def _gmm_body(gbe_ref, x_ref, w1_ref, b1_ref, w2_ref, b2_ref, eo_ref):
    x = x_ref[...]
    h = jax.nn.gelu(
        jnp.dot(
            x.astype(jnp.bfloat16),
            w1_ref[0, :, :].astype(jnp.bfloat16),
            preferred_element_type=jnp.float32,
        )
        + b1_ref[0, :, :]
    )
    eo_ref[...] = (
        jnp.dot(
            h.astype(jnp.bfloat16),
            w2_ref[0, :, :].astype(jnp.bfloat16),
            preferred_element_type=jnp.float32,
        )
        + b2_ref[0, :, :]
    )


def _final_body(
    x1_ref, g4_ref, gk_ref, w1_ref, b1_ref, w2_ref, b2_ref, g_ref, beta_ref, out_ref
):
    x = x1_ref[...]
    h = jax.nn.gelu(
        jnp.dot(
            x.astype(jnp.bfloat16),
            w1_ref[...].astype(jnp.bfloat16),
            preferred_element_type=jnp.float32,
        )
        + b1_ref[...]
    )
    fin = (
        jnp.dot(
            h.astype(jnp.bfloat16),
            w2_ref[...].astype(jnp.bfloat16),
            preferred_element_type=jnp.float32,
        )
        + b2_ref[...]
        + x
    )
    gk = gk_ref[...]
    for j in range(V * TOPK):
        fin = fin + g4_ref[j, :, :].astype(jnp.float32) * gk[:, j:j + 1]
    m = jnp.mean(fin, axis=-1, keepdims=True)
    v = jnp.mean((fin - m) ** 2, axis=-1, keepdims=True)
    out_ref[...] = (fin - m) * lax.rsqrt(v + 1e-5) * g_ref[...] + beta_ref[...]


_GCH = 32


@functools.lru_cache(maxsize=None)
def _sc_kernels():
    mesh = plsc.VectorSubcoreMesh(core_axis_name="c", subcore_axis_name="s")

    @functools.partial(
        pl.kernel,
        mesh=mesh,
        compiler_params=pltpu.CompilerParams(needs_layout_passes=False),
        out_type=jax.ShapeDtypeStruct((2 * P,), jnp.int32),
        scratch_types=[
            pltpu.VMEM((NA,), jnp.int32),
            pltpu.VMEM((SLOTS,), jnp.int32),
        ],
    )
    def sc_scatter(pos_hbm, tok_hbm, posv, tokb):
        wid = lax.axis_index("s") * _NC + lax.axis_index("c")
        base = wid * SLOTS
        pltpu.sync_copy(pos_hbm, posv)

        def init(i, carry):
            tokb[pl.ds(i * 16, 16)] = jnp.full((16,), SENT, jnp.int32)
            return carry

        lax.fori_loop(0, SLOTS // 16, init, 0)
        iota16 = lax.broadcasted_iota(jnp.int32, (16,), 0)

        def body(c, carry):
            pv = posv[pl.ds(c * 16, 16)]
            av = c * 16 + iota16
            tv = jnp.bitwise_and(av, S - 1)
            rel = pv - base
            msk = (rel >= 0) & (rel < SLOTS)
            relc = jnp.clip(rel, 0, SLOTS - 1)
            plsc.store_scatter(tokb, [relc], tv, mask=msk)
            return carry

        lax.fori_loop(0, NA // 16, body, 0)
        pltpu.sync_copy(tokb, tok_hbm.at[pl.ds(base, SLOTS)])

    NCH = SLOTS // _GCH
    NBUF = 3

    @functools.partial(
        pl.kernel,
        mesh=mesh,
        compiler_params=pltpu.CompilerParams(needs_layout_passes=False),
        out_type=jax.ShapeDtypeStruct((2 * P, D), jnp.float32),
        scratch_types=[
            pltpu.VMEM((SLOTS,), jnp.int32),
        ]
        + [pltpu.VMEM((_GCH, D), jnp.float32) for _ in range(NBUF)]
        + [pltpu.SemaphoreType.DMA for _ in range(NBUF)],
    )
    def sc_gather_x(tok_hbm, xpad_hbm, xs_hbm, tokv, r0, r1, r2, s0, s1, s2):
        bufs = (r0, r1, r2)
        sems = (s0, s1, s2)
        wid = lax.axis_index("s") * _NC + lax.axis_index("c")
        base = wid * SLOTS
        pltpu.sync_copy(tok_hbm.at[pl.ds(base, SLOTS)], tokv)

        def start(ch):
            idx = tokv.at[pl.ds(ch * _GCH, _GCH)]
            return pltpu.async_copy(
                xpad_hbm.at[idx], bufs[ch % NBUF], sems[ch % NBUF]
            )

        handles = {}
        for ch in range(min(NBUF, NCH)):
            handles[ch] = start(ch)
        for ch in range(NCH):
            handles[ch].wait()
            pltpu.sync_copy(
                bufs[ch % NBUF], xs_hbm.at[pl.ds(base + ch * _GCH, _GCH)]
            )
            nxt = ch + NBUF
            if nxt < NCH:
                handles[nxt] = start(nxt)

    @functools.partial(
        pl.kernel,
        mesh=mesh,
        compiler_params=pltpu.CompilerParams(needs_layout_passes=False),
        out_type=jax.ShapeDtypeStruct((V * TOPK * S, D), jnp.float32),
        scratch_types=[
            pltpu.VMEM((_GCH,), jnp.int32),
            pltpu.VMEM((_GCH,), jnp.int32),
            pltpu.VMEM((_GCH, D), jnp.float32),
            pltpu.VMEM((_GCH, D), jnp.float32),
            pltpu.SemaphoreType.DMA,
            pltpu.SemaphoreType.DMA,
        ],
    )
    def sc_gather_eo(pos_hbm, eo_hbm, g4_hbm, p0, p1, r0, r1, s0, s1):
        pv = (p0, p1)
        bufs = (r0, r1)
        sems = (s0, s1)
        wid = lax.axis_index("s") * _NC + lax.axis_index("c")
        tbase = wid * TT
        nch = TT // _GCH
        total = V * TOPK * nch

        def start(ci):
            j, half = ci // nch, ci % nch
            off = j * S + tbase + half * _GCH
            pltpu.sync_copy(pos_hbm.at[pl.ds(off, _GCH)], pv[ci % 2])
            return pltpu.async_copy(eo_hbm.at[pv[ci % 2]], bufs[ci % 2], sems[ci % 2])

        handles = {0: start(0), 1: start(1)}
        for ci in range(total):
            handles[ci].wait()
            j, half = ci // nch, ci % nch
            off = j * S + tbase + half * _GCH
            pltpu.sync_copy(bufs[ci % 2], g4_hbm.at[pl.ds(off, _GCH)])
            nxt = ci + 2
            if nxt < total:
                handles[nxt] = start(nxt)

    return sc_scatter, sc_gather_x, sc_gather_eo


def kernel(x, total_logits, total_masks, attn_mask, Wq, bq, Wk, bk, Wv, bv, Wo, bo,
           g1, beta1, g2, beta2, W1v, b1v, W2v, b2v, W1g, b1g, W2g, b2g):
    f32 = jnp.float32
    xf = x.reshape(S, D)

    lg = total_logits.reshape(VS, E)
    mk = total_masks.reshape(VS, E)
    posk, gatek, gbe2, guide2 = pl.pallas_call(
        _route_body,
        in_specs=[
            pl.BlockSpec((VS, E), lambda: (0, 0)),
            pl.BlockSpec((VS, E), lambda: (0, 0)),
        ],
        out_specs=[
            pl.BlockSpec((VS, TOPK), lambda: (0, 0)),
            pl.BlockSpec((VS, TOPK), lambda: (0, 0)),
            pl.BlockSpec((NBT, 1), lambda: (0, 0)),
            pl.BlockSpec((1, 1), lambda: (0, 0)),
        ],
        out_shape=[
            jax.ShapeDtypeStruct((VS, TOPK), jnp.int32),
            jax.ShapeDtypeStruct((VS, TOPK), f32),
            jax.ShapeDtypeStruct((NBT, 1), jnp.int32),
            jax.ShapeDtypeStruct((1, 1), f32),
        ],
    )(lg, mk)
    total_guide = guide2[0, 0]
    pos4 = posk.reshape(V, S, TOPK).transpose(0, 2, 1).reshape(NA)
    gate4 = gatek.reshape(V, S, TOPK).transpose(1, 0, 2).reshape(S, V * TOPK)
    gbe = gbe2.reshape(NBT)

    sc_scatter, sc_gather_x, sc_gather_eo = _sc_kernels()
    tokbuf = sc_scatter(pos4)

    Wqkv = jnp.concatenate([Wq, Wk, Wv], axis=1)
    bqkv = jnp.concatenate([bq, bk, bv]).reshape(1, 3 * D)
    qkv = pl.pallas_call(
        _qkv_body,
        grid=(S // BT,),
        in_specs=[
            pl.BlockSpec((BT, D), lambda i: (i, 0)),
            pl.BlockSpec((D, 3 * D), lambda i: (0, 0)),
            pl.BlockSpec((1, 3 * D), lambda i: (0, 0)),
        ],
        out_specs=pl.BlockSpec((BT, 3 * D), lambda i: (i, 0)),
        out_shape=jax.ShapeDtypeStruct((S, 3 * D), jnp.bfloat16),
    )(xf, Wqkv, bqkv)

    q = qkv[:, :D].reshape(S, H, DH).transpose(1, 0, 2)
    k = qkv[:, D:2 * D].reshape(S, H, DH).transpose(1, 0, 2)
    v = qkv[:, 2 * D:].reshape(S, H, DH).transpose(1, 0, 2)

    o = pl.pallas_call(
        _attn_body,
        grid=(H, S // BQ),
        in_specs=[
            pl.BlockSpec((1, BQ, DH), lambda h, i: (h, i, 0)),
            pl.BlockSpec((1, S, DH), lambda h, i: (h, 0, 0)),
            pl.BlockSpec((1, S, DH), lambda h, i: (h, 0, 0)),
        ],
        out_specs=pl.BlockSpec((1, BQ, DH), lambda h, i: (h, i, 0)),
        out_shape=jax.ShapeDtypeStruct((H, S, DH), jnp.bfloat16),
    )(q, k, v)
    o2 = o.transpose(1, 0, 2).reshape(S, D)

    x1 = pl.pallas_call(
        _oproj_ln_body,
        grid=(S // BT,),
        in_specs=[
            pl.BlockSpec((BT, D), lambda i: (i, 0)),
            pl.BlockSpec((D, D), lambda i: (0, 0)),
            pl.BlockSpec((1, D), lambda i: (0, 0)),
            pl.BlockSpec((BT, D), lambda i: (i, 0)),
            pl.BlockSpec((1, D), lambda i: (0, 0)),
            pl.BlockSpec((1, D), lambda i: (0, 0)),
        ],
        out_specs=pl.BlockSpec((BT, D), lambda i: (i, 0)),
        out_shape=jax.ShapeDtypeStruct((S, D), f32),
    )(o2, Wo, bo.reshape(1, D), xf, g1.reshape(1, D), beta1.reshape(1, D))

    xpad = jnp.concatenate([x1, jnp.zeros((8, D), f32)], axis=0)
    xs = sc_gather_x(tokbuf, xpad)

    W1r = W1v.reshape(V * E, D, DFF)
    b1r = b1v.reshape(V * E, 1, DFF)
    W2r = W2v.reshape(V * E, DFF, D)
    b2r = b2v.reshape(V * E, 1, D)
    eo = pl.pallas_call(
        _gmm_body,
        grid_spec=pltpu.PrefetchScalarGridSpec(
            num_scalar_prefetch=1,
            grid=(NBT,),
            in_specs=[
                pl.BlockSpec((G, D), lambda i, gbe_r: (i, 0)),
                pl.BlockSpec((1, D, DFF), lambda i, gbe_r: (gbe_r[i], 0, 0)),
                pl.BlockSpec((1, 1, DFF), lambda i, gbe_r: (gbe_r[i], 0, 0)),
                pl.BlockSpec((1, DFF, D), lambda i, gbe_r: (gbe_r[i], 0, 0)),
                pl.BlockSpec((1, 1, D), lambda i, gbe_r: (gbe_r[i], 0, 0)),
            ],
            out_specs=pl.BlockSpec((G, D), lambda i, gbe_r: (i, 0)),
        ),
        out_shape=jax.ShapeDtypeStruct((2 * P, D), f32),
    )(gbe, xs, W1r, b1r, W2r, b2r)

    g4 = sc_gather_eo(pos4, eo).reshape(V * TOPK, S, D)

    out = pl.pallas_call(
        _final_body,
        grid=(S // BT,),
        in_specs=[
            pl.BlockSpec((BT, D), lambda i: (i, 0)),
            pl.BlockSpec((V * TOPK, BT, D), lambda i: (0, i, 0)),
            pl.BlockSpec((BT, V * TOPK), lambda i: (i, 0)),
            pl.BlockSpec((D, DFFG), lambda i: (0, 0)),
            pl.BlockSpec((1, DFFG), lambda i: (0, 0)),
            pl.BlockSpec((DFFG, D), lambda i: (0, 0)),
            pl.BlockSpec((1, D), lambda i: (0, 0)),
            pl.BlockSpec((1, D), lambda i: (0, 0)),
            pl.BlockSpec((1, D), lambda i: (0, 0)),
        ],
        out_specs=pl.BlockSpec((BT, D), lambda i: (i, 0)),
        out_shape=jax.ShapeDtypeStruct((S, D), f32),
    )(
        x1, g4, gate4, W1g, b1g.reshape(1, DFFG), W2g, b2g.reshape(1, D),
        g2.reshape(1, D), beta2.reshape(1, D),
    )

    return out.reshape(B, S, D), total_guide


# speedup vs baseline: 1.9650x; 1.8176x over previous
import functools
import math

import jax
import jax.numpy as jnp
from jax import lax
from jax.experimental import pallas as pl
from jax.experimental.pallas import tpu as pltpu
from jax.experimental.pallas import tpu_sc as plsc

B, S, D, H = 1, 2048, 1024, 16
V, E, TOPK = 2, 8, 2
DFF, DFFG = 1024, 2048
DH = D // H
DP = D // 2

BT = 256
BQ = 256
G = 256
P = 6144
NB = P // G
NBT = V * NB
VS = V * S
NA = V * S * TOPK

_NC, _NS = 2, 16
NW = _NC * _NS
SLOTS = 2 * P // NW
TT = S // NW
SENT = 0


def _qkv_body(x_ref, w_ref, b_ref, o_ref):
    acc = jnp.dot(
        x_ref[...].astype(jnp.bfloat16),
        w_ref[...].astype(jnp.bfloat16),
        preferred_element_type=jnp.float32,
    )
    o_ref[...] = (acc + b_ref[...]).astype(jnp.bfloat16)


def _attn_body(q_ref, k_ref, v_ref, o_ref, *, qoff):
    qi = pl.program_id(1)
    skv = k_ref.shape[0]
    for h2 in range(2):
        cs = h2 * DH
        q = q_ref[:, cs:cs + DH]
        k = k_ref[:, cs:cs + DH]
        v = v_ref[:, cs:cs + DH]
        sc = lax.dot_general(
            q, k, (((1,), (1,)), ((), ())), preferred_element_type=jnp.float32
        ) / math.sqrt(DH)
        w = skv - qoff
        rows = qi * BQ + lax.broadcasted_iota(jnp.int32, (BQ, w), 0)
        cols = lax.broadcasted_iota(jnp.int32, (BQ, w), 1)
        tail = jnp.where(cols > rows, jnp.float32(-1e9), sc[:, qoff:])
        if qoff:
            sc = jnp.concatenate([sc[:, :qoff], tail], axis=1)
        else:
            sc = tail
        pr = jax.nn.softmax(sc, axis=-1)
        o_ref[:, cs:cs + DH] = jnp.dot(
            pr.astype(jnp.bfloat16), v, preferred_element_type=jnp.float32
        ).astype(jnp.bfloat16)


def _oproj_ln_body(o_ref, w_ref, b_ref, x_ref, g_ref, beta_ref, out_ref):
    y = (
        jnp.dot(
            o_ref[...],
            w_ref[...].astype(jnp.bfloat16),
            preferred_element_type=jnp.float32,
        )
        + b_ref[...]
        + x_ref[...]
    )
    m = jnp.mean(y, axis=-1, keepdims=True)
    v = jnp.mean((y - m) ** 2, axis=-1, keepdims=True)
    out_ref[...] = (y - m) * lax.rsqrt(v + 1e-5) * g_ref[...] + beta_ref[...]


def _cumsum_rows(x):
    n = x.shape[0]
    s = 1
    while s < n:
        x = x + jnp.concatenate(
            [jnp.zeros((s, x.shape[1]), x.dtype), x[:-s, :]], axis=0
        )
        s *= 2
    return x


def _route_body(lg_ref, mk_ref, posk_ref, gatek_ref, gbe_ref, act_ref, guide_ref):
    lg = lg_ref[...]
    probs = jax.nn.softmax(lg, axis=-1)
    iota_e = lax.broadcasted_iota(jnp.int32, (VS, E), 1)
    m1 = jnp.max(probs, axis=-1, keepdims=True)
    i1 = jnp.min(jnp.where(probs == m1, iota_e, E), axis=-1, keepdims=True)
    oh1 = iota_e == i1
    p2 = jnp.where(oh1, jnp.float32(-1.0), probs)
    m2 = jnp.max(p2, axis=-1, keepdims=True)
    i2 = jnp.min(jnp.where(p2 == m2, iota_e, E), axis=-1, keepdims=True)
    oh2 = iota_e == i2
    ssum = m1 + m2
    gatek_ref[...] = jnp.concatenate([m1 / ssum, m2 / ssum], axis=1)

    mk = mk_ref[...]
    mn = mk / (jnp.sum(mk, axis=-1, keepdims=True) + 1e-9)
    guide_ref[...] = (-jnp.sum(mn * jnp.log(probs + 1e-9)) / (S * V)).reshape(1, 1)

    cnt = (oh1 | oh2).astype(jnp.int32)
    C = _cumsum_rows(cnt)
    n0 = C[S - 1:S, :]
    n1 = C[VS - 1:VS, :] - n0
    rows = lax.broadcasted_iota(jnp.int32, (VS, 1), 0)
    is_v1 = rows >= S
    excl = C - cnt - jnp.where(is_v1, 1, 0) * n0
    np0 = ((n0 + (G - 1)) // G) * G
    np1 = ((n1 + (G - 1)) // G) * G
    tri = (
        lax.broadcasted_iota(jnp.int32, (E, E), 0)
        < lax.broadcasted_iota(jnp.int32, (E, E), 1)
    ).astype(jnp.float32)
    po0 = jnp.dot(
        np0.astype(jnp.float32), tri, preferred_element_type=jnp.float32
    ).astype(jnp.int32)
    po1 = jnp.dot(
        np1.astype(jnp.float32), tri, preferred_element_type=jnp.float32
    ).astype(jnp.int32)
    po_full = jnp.where(is_v1, po1, po0)
    base = po_full + excl + jnp.where(is_v1, P, 0)
    pos0 = jnp.sum(jnp.where(oh1, base, 0), axis=-1, keepdims=True)
    pos1 = jnp.sum(jnp.where(oh2, base, 0), axis=-1, keepdims=True)
    posk_ref[...] = jnp.concatenate([pos0, pos1], axis=1)

    r = lax.broadcasted_iota(jnp.int32, (NBT, 1), 0)
    isb1 = r >= NB
    iloc = jnp.where(isb1, r - NB, r)
    ends = jnp.where(isb1, po1 + np1, po0 + np0)
    cntb = jnp.sum((ends <= iloc * G).astype(jnp.int32), axis=-1, keepdims=True)
    gbe_ref[...] = jnp.minimum(cntb, E - 1) + jnp.where(isb1, E, 0)
    act_ref[...] = (iloc * G < ends[:, E - 1:E]).astype(jnp.int32)


def _gmm_body(gbe_ref, act_ref, tok_ref, xb_ref, w1_ref, b1_ref, w2_ref, b2_ref,
              eo_ref):
    @pl.when(act_ref[pl.program_id(0)] == 1)
    def _():
        _gmm_inner(tok_ref, xb_ref, w1_ref, b1_ref, w2_ref, b2_ref, eo_ref)


def _gmm_inner(tok_ref, xb_ref, w1_ref, b1_ref, w2_ref, b2_ref, eo_ref):
    tids = tok_ref[0, 0, :]
    onehot = (
        tids[:, None] == lax.broadcasted_iota(jnp.int32, (G, S), 1)
    ).astype(jnp.bfloat16)
    xg = jnp.dot(onehot, xb_ref[...], preferred_element_type=jnp.float32)
    h = jax.nn.gelu(
        jnp.dot(
            xg.astype(jnp.bfloat16),
            w1_ref[0, :, :].astype(jnp.bfloat16),
            preferred_element_type=jnp.float32,
        )
        + b1_ref[0, :, :]
    )
    eo = (
        jnp.dot(
            h.astype(jnp.bfloat16),
            w2_ref[0, :, :].astype(jnp.bfloat16),
            preferred_element_type=jnp.float32,
        )
        + b2_ref[0, :, :]
    )
    ai = lax.bitcast_convert_type(eo[:, :DP], jnp.int32)
    bi = lax.bitcast_convert_type(eo[:, DP:], jnp.int32)
    eo_ref[...] = lax.shift_right_logical(ai, 16) | (bi & jnp.int32(-65536))


def _gffn_body(x1_ref, w1_ref, b1_ref, w2_ref, b2_ref, out_ref):
    x = x1_ref[...]
    h = jax.nn.gelu(
        jnp.dot(
            x.astype(jnp.bfloat16),
            w1_ref[...].astype(jnp.bfloat16),
            preferred_element_type=jnp.float32,
        )
        + b1_ref[...]
    )
    out_ref[...] = (
        jnp.dot(
            h.astype(jnp.bfloat16),
            w2_ref[...].astype(jnp.bfloat16),
            preferred_element_type=jnp.float32,
        )
        + b2_ref[...]
    )


def _final_body(x1_ref, gen_ref, g4_ref, gk_ref, g_ref, beta_ref, out_ref):
    x = x1_ref[...]
    gen = gen_ref[...]
    gk = gk_ref[...]
    lo = gen[:, :DP] + x[:, :DP]
    hi = gen[:, DP:] + x[:, DP:]
    for j in range(V * TOPK):
        w = g4_ref[j, :, :]
        a = lax.bitcast_convert_type(lax.shift_left(w, 16), jnp.float32)
        b = lax.bitcast_convert_type(w & jnp.int32(-65536), jnp.float32)
        gj = gk[:, j:j + 1]
        lo = lo + a * gj
        hi = hi + b * gj
    fin = jnp.concatenate([lo, hi], axis=1)
    m = jnp.mean(fin, axis=-1, keepdims=True)
    v = jnp.mean((fin - m) ** 2, axis=-1, keepdims=True)
    out_ref[...] = (fin - m) * lax.rsqrt(v + 1e-5) * g_ref[...] + beta_ref[...]


_GCH = 32


@functools.lru_cache(maxsize=None)
def _sc_kernels():
    mesh = plsc.VectorSubcoreMesh(core_axis_name="c", subcore_axis_name="s")

    @functools.partial(
        pl.kernel,
        mesh=mesh,
        compiler_params=pltpu.CompilerParams(needs_layout_passes=False),
        out_type=jax.ShapeDtypeStruct((2 * P,), jnp.int32),
        scratch_types=[
            pltpu.VMEM((NA,), jnp.int32),
            pltpu.VMEM((SLOTS,), jnp.int32),
        ],
    )
    def sc_scatter(pos_hbm, tok_hbm, posv, tokb):
        wid = lax.axis_index("s") * _NC + lax.axis_index("c")
        base = wid * SLOTS
        pltpu.sync_copy(pos_hbm, posv)

        def init(i, carry):
            tokb[pl.ds(i * 16, 16)] = jnp.full((16,), SENT, jnp.int32)
            return carry

        lax.fori_loop(0, SLOTS // 16, init, 0)
        iota16 = lax.broadcasted_iota(jnp.int32, (16,), 0)

        def body(c, carry):
            pv = posv[pl.ds(c * 16, 16)]
            av = c * 16 + iota16
            tv = jnp.bitwise_and(av, S - 1)
            rel = pv - base
            msk = (rel >= 0) & (rel < SLOTS)
            relc = jnp.clip(rel, 0, SLOTS - 1)
            plsc.store_scatter(tokb, [relc], tv, mask=msk)
            return carry

        lax.fori_loop(0, NA // 16, body, 0)
        pltpu.sync_copy(tokb, tok_hbm.at[pl.ds(base, SLOTS)])

    @functools.partial(
        pl.kernel,
        mesh=mesh,
        compiler_params=pltpu.CompilerParams(needs_layout_passes=False),
        out_type=jax.ShapeDtypeStruct((V * TOPK * S, DP), jnp.int32),
        scratch_types=[
            pltpu.VMEM((_GCH,), jnp.int32),
            pltpu.VMEM((_GCH,), jnp.int32),
            pltpu.VMEM((_GCH, DP), jnp.int32),
            pltpu.VMEM((_GCH, DP), jnp.int32),
            pltpu.SemaphoreType.DMA,
            pltpu.SemaphoreType.DMA,
        ],
    )
    def sc_gather_eo(pos_hbm, eo_hbm, g4_hbm, p0, p1, r0, r1, s0, s1):
        pv = (p0, p1)
        bufs = (r0, r1)
        sems = (s0, s1)
        wid = lax.axis_index("s") * _NC + lax.axis_index("c")
        tbase = wid * TT
        nch = TT // _GCH
        total = V * TOPK * nch

        def start(ci):
            j, half = ci // nch, ci % nch
            off = j * S + tbase + half * _GCH
            pltpu.sync_copy(pos_hbm.at[pl.ds(off, _GCH)], pv[ci % 2])
            return pltpu.async_copy(eo_hbm.at[pv[ci % 2]], bufs[ci % 2], sems[ci % 2])

        handles = {0: start(0), 1: start(1)}
        for ci in range(total):
            handles[ci].wait()
            j, half = ci // nch, ci % nch
            off = j * S + tbase + half * _GCH
            pltpu.sync_copy(bufs[ci % 2], g4_hbm.at[pl.ds(off, _GCH)])
            nxt = ci + 2
            if nxt < total:
                handles[nxt] = start(nxt)

    return sc_scatter, sc_gather_eo


def kernel(x, total_logits, total_masks, attn_mask, Wq, bq, Wk, bk, Wv, bv, Wo, bo,
           g1, beta1, g2, beta2, W1v, b1v, W2v, b2v, W1g, b1g, W2g, b2g):
    f32 = jnp.float32
    xf = x.reshape(S, D)

    lg = total_logits.reshape(VS, E)
    mk = total_masks.reshape(VS, E)
    posk, gatek, gbe2, act2, guide2 = pl.pallas_call(
        _route_body,
        in_specs=[
            pl.BlockSpec((VS, E), lambda: (0, 0)),
            pl.BlockSpec((VS, E), lambda: (0, 0)),
        ],
        out_specs=[
            pl.BlockSpec((VS, TOPK), lambda: (0, 0)),
            pl.BlockSpec((VS, TOPK), lambda: (0, 0)),
            pl.BlockSpec((NBT, 1), lambda: (0, 0)),
            pl.BlockSpec((NBT, 1), lambda: (0, 0)),
            pl.BlockSpec((1, 1), lambda: (0, 0)),
        ],
        out_shape=[
            jax.ShapeDtypeStruct((VS, TOPK), jnp.int32),
            jax.ShapeDtypeStruct((VS, TOPK), f32),
            jax.ShapeDtypeStruct((NBT, 1), jnp.int32),
            jax.ShapeDtypeStruct((NBT, 1), jnp.int32),
            jax.ShapeDtypeStruct((1, 1), f32),
        ],
    )(lg, mk)
    total_guide = guide2[0, 0]
    pos4 = posk.reshape(V, S, TOPK).transpose(0, 2, 1).reshape(NA)
    gate4 = gatek.reshape(V, S, TOPK).transpose(1, 0, 2).reshape(S, V * TOPK)
    gbe = gbe2.reshape(NBT)

    sc_scatter, sc_gather_eo = _sc_kernels()
    tokbuf = sc_scatter(pos4)

    Wqkv = jnp.concatenate([Wq, Wk, Wv], axis=1)
    bqkv = jnp.concatenate([bq, bk, bv]).reshape(1, 3 * D)
    qkv = pl.pallas_call(
        _qkv_body,
        grid=(S // BT,),
        in_specs=[
            pl.BlockSpec((BT, D), lambda i: (i, 0)),
            pl.BlockSpec((D, 3 * D), lambda i: (0, 0)),
            pl.BlockSpec((1, 3 * D), lambda i: (0, 0)),
        ],
        out_specs=pl.BlockSpec((BT, 3 * D), lambda i: (i, 0)),
        out_shape=jax.ShapeDtypeStruct((S, 3 * D), jnp.bfloat16),
    )(xf, Wqkv, bqkv)

    QCH = 512
    o_parts = []
    for t in range(S // QCH):
        skv = (t + 1) * QCH
        qoff = t * QCH
        o_parts.append(pl.pallas_call(
            functools.partial(_attn_body, qoff=qoff),
            grid=(H // 2, QCH // BQ),
            in_specs=[
                pl.BlockSpec((BQ, 2 * DH), lambda hh, i, t=t: (t * QCH // BQ + i, hh)),
                pl.BlockSpec((skv, 2 * DH), lambda hh, i: (0, H // 2 + hh)),
                pl.BlockSpec((skv, 2 * DH), lambda hh, i: (0, H + hh)),
            ],
            out_specs=pl.BlockSpec((BQ, 2 * DH), lambda hh, i: (i, hh)),
            out_shape=jax.ShapeDtypeStruct((QCH, D), jnp.bfloat16),
        )(qkv, qkv, qkv))
    o2 = jnp.concatenate(o_parts, axis=0)

    x1 = pl.pallas_call(
        _oproj_ln_body,
        grid=(S // BT,),
        in_specs=[
            pl.BlockSpec((BT, D), lambda i: (i, 0)),
            pl.BlockSpec((D, D), lambda i: (0, 0)),
            pl.BlockSpec((1, D), lambda i: (0, 0)),
            pl.BlockSpec((BT, D), lambda i: (i, 0)),
            pl.BlockSpec((1, D), lambda i: (0, 0)),
            pl.BlockSpec((1, D), lambda i: (0, 0)),
        ],
        out_specs=pl.BlockSpec((BT, D), lambda i: (i, 0)),
        out_shape=jax.ShapeDtypeStruct((S, D), f32),
    )(o2, Wo, bo.reshape(1, D), xf, g1.reshape(1, D), beta1.reshape(1, D))

    tok3 = tokbuf.reshape(NBT, 1, G)
    xb = x1.astype(jnp.bfloat16)
    W1r = W1v.reshape(V * E, D, DFF)
    b1r = b1v.reshape(V * E, 1, DFF)
    W2r = W2v.reshape(V * E, DFF, D)
    b2r = b2v.reshape(V * E, 1, D)
    eo = pl.pallas_call(
        _gmm_body,
        grid_spec=pltpu.PrefetchScalarGridSpec(
            num_scalar_prefetch=2,
            grid=(NBT,),
            in_specs=[
                pl.BlockSpec((1, 1, G), lambda i, gbe_r, act_r: (i, 0, 0)),
                pl.BlockSpec((S, D), lambda i, gbe_r, act_r: (0, 0)),
                pl.BlockSpec((1, D, DFF), lambda i, gbe_r, act_r: (gbe_r[i], 0, 0)),
                pl.BlockSpec((1, 1, DFF), lambda i, gbe_r, act_r: (gbe_r[i], 0, 0)),
                pl.BlockSpec((1, DFF, D), lambda i, gbe_r, act_r: (gbe_r[i], 0, 0)),
                pl.BlockSpec((1, 1, D), lambda i, gbe_r, act_r: (gbe_r[i], 0, 0)),
            ],
            out_specs=pl.BlockSpec((G, DP), lambda i, gbe_r, act_r: (i, 0)),
        ),
        out_shape=jax.ShapeDtypeStruct((2 * P, DP), jnp.int32),
    )(gbe, act2.reshape(NBT), tok3, xb, W1r, b1r, W2r, b2r)

    g4 = sc_gather_eo(pos4, eo).reshape(V * TOPK, S, DP)

    gen = pl.pallas_call(
        _gffn_body,
        grid=(S // BT,),
        in_specs=[
            pl.BlockSpec((BT, D), lambda i: (i, 0)),
            pl.BlockSpec((D, DFFG), lambda i: (0, 0)),
            pl.BlockSpec((1, DFFG), lambda i: (0, 0)),
            pl.BlockSpec((DFFG, D), lambda i: (0, 0)),
            pl.BlockSpec((1, D), lambda i: (0, 0)),
        ],
        out_specs=pl.BlockSpec((BT, D), lambda i: (i, 0)),
        out_shape=jax.ShapeDtypeStruct((S, D), f32),
    )(x1, W1g, b1g.reshape(1, DFFG), W2g, b2g.reshape(1, D))

    out = pl.pallas_call(
        _final_body,
        grid=(S // BT,),
        in_specs=[
            pl.BlockSpec((BT, D), lambda i: (i, 0)),
            pl.BlockSpec((BT, D), lambda i: (i, 0)),
            pl.BlockSpec((V * TOPK, BT, DP), lambda i: (0, i, 0)),
            pl.BlockSpec((BT, V * TOPK), lambda i: (i, 0)),
            pl.BlockSpec((1, D), lambda i: (0, 0)),
            pl.BlockSpec((1, D), lambda i: (0, 0)),
        ],
        out_specs=pl.BlockSpec((BT, D), lambda i: (i, 0)),
        out_shape=jax.ShapeDtypeStruct((S, D), f32),
    )(x1, gen, g4, gate4, g2.reshape(1, D), beta2.reshape(1, D))

    return out.reshape(B, S, D), total_guide


# speedup vs baseline: 2.0505x; 1.0435x over previous
import functools
import math

import jax
import jax.numpy as jnp
from jax import lax
from jax.experimental import pallas as pl
from jax.experimental.pallas import tpu as pltpu
from jax.experimental.pallas import tpu_sc as plsc

B, S, D, H = 1, 2048, 1024, 16
V, E, TOPK = 2, 8, 2
DFF, DFFG = 1024, 2048
DH = D // H
DP = D // 2

BT = 256
BQ = 512
G = 256
P = 6144
NB = P // G
NBT = V * NB
VS = V * S
NA = V * S * TOPK

_NC, _NS = 2, 16
NW = _NC * _NS
SLOTS = 2 * P // NW
TT = S // NW
SENT = 0


def _qkv_body(x_ref, w_ref, b_ref, o_ref):
    acc = jnp.dot(
        x_ref[...].astype(jnp.bfloat16),
        w_ref[...].astype(jnp.bfloat16),
        preferred_element_type=jnp.float32,
    )
    o_ref[...] = (acc + b_ref[...]).astype(jnp.bfloat16)


def _attn_body(q_ref, k_ref, v_ref, o_ref, *, qoff):
    qi = pl.program_id(1)
    skv = k_ref.shape[0]
    for h2 in range(2):
        cs = h2 * DH
        q = q_ref[:, cs:cs + DH]
        k = k_ref[:, cs:cs + DH]
        v = v_ref[:, cs:cs + DH]
        sc = lax.dot_general(
            q, k, (((1,), (1,)), ((), ())), preferred_element_type=jnp.float32
        ) / math.sqrt(DH)
        w = skv - qoff
        rows = qi * BQ + lax.broadcasted_iota(jnp.int32, (BQ, w), 0)
        cols = lax.broadcasted_iota(jnp.int32, (BQ, w), 1)
        tail = jnp.where(cols > rows, jnp.float32(-1e9), sc[:, qoff:])
        if qoff:
            sc = jnp.concatenate([sc[:, :qoff], tail], axis=1)
        else:
            sc = tail
        pr = jax.nn.softmax(sc, axis=-1)
        o_ref[:, cs:cs + DH] = jnp.dot(
            pr.astype(jnp.bfloat16), v, preferred_element_type=jnp.float32
        ).astype(jnp.bfloat16)


def _oproj_ln_body(o_ref, w_ref, b_ref, x_ref, g_ref, beta_ref, out_ref):
    y = (
        jnp.dot(
            o_ref[...],
            w_ref[...].astype(jnp.bfloat16),
            preferred_element_type=jnp.float32,
        )
        + b_ref[...]
        + x_ref[...]
    )
    m = jnp.mean(y, axis=-1, keepdims=True)
    v = jnp.mean((y - m) ** 2, axis=-1, keepdims=True)
    out_ref[...] = (y - m) * lax.rsqrt(v + 1e-5) * g_ref[...] + beta_ref[...]


def _cumsum_rows(x):
    n = x.shape[0]
    s = 1
    while s < n:
        x = x + jnp.concatenate(
            [jnp.zeros((s, x.shape[1]), x.dtype), x[:-s, :]], axis=0
        )
        s *= 2
    return x


def _route_body(lg_ref, mk_ref, posk_ref, gatek_ref, gbe_ref, act_ref, guide_ref):
    lg = lg_ref[...]
    probs = jax.nn.softmax(lg, axis=-1)
    iota_e = lax.broadcasted_iota(jnp.int32, (VS, E), 1)
    m1 = jnp.max(probs, axis=-1, keepdims=True)
    i1 = jnp.min(jnp.where(probs == m1, iota_e, E), axis=-1, keepdims=True)
    oh1 = iota_e == i1
    p2 = jnp.where(oh1, jnp.float32(-1.0), probs)
    m2 = jnp.max(p2, axis=-1, keepdims=True)
    i2 = jnp.min(jnp.where(p2 == m2, iota_e, E), axis=-1, keepdims=True)
    oh2 = iota_e == i2
    ssum = m1 + m2
    gatek_ref[...] = jnp.concatenate([m1 / ssum, m2 / ssum], axis=1)

    mk = mk_ref[...]
    mn = mk / (jnp.sum(mk, axis=-1, keepdims=True) + 1e-9)
    guide_ref[...] = (-jnp.sum(mn * jnp.log(probs + 1e-9)) / (S * V)).reshape(1, 1)

    cnt = (oh1 | oh2).astype(jnp.int32)
    C = _cumsum_rows(cnt)
    n0 = C[S - 1:S, :]
    n1 = C[VS - 1:VS, :] - n0
    rows = lax.broadcasted_iota(jnp.int32, (VS, 1), 0)
    is_v1 = rows >= S
    excl = C - cnt - jnp.where(is_v1, 1, 0) * n0
    np0 = ((n0 + (G - 1)) // G) * G
    np1 = ((n1 + (G - 1)) // G) * G
    tri = (
        lax.broadcasted_iota(jnp.int32, (E, E), 0)
        < lax.broadcasted_iota(jnp.int32, (E, E), 1)
    ).astype(jnp.float32)
    po0 = jnp.dot(
        np0.astype(jnp.float32), tri, preferred_element_type=jnp.float32
    ).astype(jnp.int32)
    po1 = jnp.dot(
        np1.astype(jnp.float32), tri, preferred_element_type=jnp.float32
    ).astype(jnp.int32)
    po_full = jnp.where(is_v1, po1, po0)
    base = po_full + excl + jnp.where(is_v1, P, 0)
    pos0 = jnp.sum(jnp.where(oh1, base, 0), axis=-1, keepdims=True)
    pos1 = jnp.sum(jnp.where(oh2, base, 0), axis=-1, keepdims=True)
    posk_ref[...] = jnp.concatenate([pos0, pos1], axis=1)

    r = lax.broadcasted_iota(jnp.int32, (NBT, 1), 0)
    isb1 = r >= NB
    iloc = jnp.where(isb1, r - NB, r)
    ends = jnp.where(isb1, po1 + np1, po0 + np0)
    cntb = jnp.sum((ends <= iloc * G).astype(jnp.int32), axis=-1, keepdims=True)
    gbe_ref[...] = jnp.minimum(cntb, E - 1) + jnp.where(isb1, E, 0)
    act_ref[...] = (iloc * G < ends[:, E - 1:E]).astype(jnp.int32)


def _gmm_body(gbe_ref, act_ref, tok_ref, xb_ref, w1_ref, b1_ref, w2_ref, b2_ref,
              eo_ref):
    @pl.when(act_ref[pl.program_id(0)] == 1)
    def _():
        _gmm_inner(tok_ref, xb_ref, w1_ref, b1_ref, w2_ref, b2_ref, eo_ref)


def _gmm_inner(tok_ref, xb_ref, w1_ref, b1_ref, w2_ref, b2_ref, eo_ref):
    tids = tok_ref[0, 0, :]
    onehot = (
        tids[:, None] == lax.broadcasted_iota(jnp.int32, (G, S), 1)
    ).astype(jnp.bfloat16)
    xg = jnp.dot(onehot, xb_ref[...], preferred_element_type=jnp.float32)
    h = jax.nn.gelu(
        jnp.dot(
            xg.astype(jnp.bfloat16),
            w1_ref[0, :, :].astype(jnp.bfloat16),
            preferred_element_type=jnp.float32,
        )
        + b1_ref[0, :, :]
    )
    eo = (
        jnp.dot(
            h.astype(jnp.bfloat16),
            w2_ref[0, :, :].astype(jnp.bfloat16),
            preferred_element_type=jnp.float32,
        )
        + b2_ref[0, :, :]
    )
    ai = lax.bitcast_convert_type(eo[:, :DP], jnp.int32)
    bi = lax.bitcast_convert_type(eo[:, DP:], jnp.int32)
    eo_ref[...] = lax.shift_right_logical(ai, 16) | (bi & jnp.int32(-65536))


def _gffn_body(x1_ref, w1_ref, b1_ref, w2_ref, b2_ref, out_ref):
    x = x1_ref[...]
    h = jax.nn.gelu(
        jnp.dot(
            x.astype(jnp.bfloat16),
            w1_ref[...].astype(jnp.bfloat16),
            preferred_element_type=jnp.float32,
        )
        + b1_ref[...]
    )
    out_ref[...] = (
        jnp.dot(
            h.astype(jnp.bfloat16),
            w2_ref[...].astype(jnp.bfloat16),
            preferred_element_type=jnp.float32,
        )
        + b2_ref[...]
    )


def _final_body(x1_ref, gen_ref, g4_ref, gk_ref, g_ref, beta_ref, out_ref):
    x = x1_ref[...]
    gen = gen_ref[...]
    gk = gk_ref[...]
    lo = gen[:, :DP] + x[:, :DP]
    hi = gen[:, DP:] + x[:, DP:]
    for j in range(V * TOPK):
        w = g4_ref[j, :, :]
        a = lax.bitcast_convert_type(lax.shift_left(w, 16), jnp.float32)
        b = lax.bitcast_convert_type(w & jnp.int32(-65536), jnp.float32)
        gj = gk[:, j:j + 1]
        lo = lo + a * gj
        hi = hi + b * gj
    fin = jnp.concatenate([lo, hi], axis=1)
    m = jnp.mean(fin, axis=-1, keepdims=True)
    v = jnp.mean((fin - m) ** 2, axis=-1, keepdims=True)
    out_ref[...] = (fin - m) * lax.rsqrt(v + 1e-5) * g_ref[...] + beta_ref[...]


_GCH = 32


@functools.lru_cache(maxsize=None)
def _sc_kernels():
    mesh = plsc.VectorSubcoreMesh(core_axis_name="c", subcore_axis_name="s")

    @functools.partial(
        pl.kernel,
        mesh=mesh,
        compiler_params=pltpu.CompilerParams(needs_layout_passes=False),
        out_type=jax.ShapeDtypeStruct((2 * P,), jnp.int32),
        scratch_types=[
            pltpu.VMEM((NA,), jnp.int32),
            pltpu.VMEM((SLOTS,), jnp.int32),
        ],
    )
    def sc_scatter(pos_hbm, tok_hbm, posv, tokb):
        wid = lax.axis_index("s") * _NC + lax.axis_index("c")
        base = wid * SLOTS
        pltpu.sync_copy(pos_hbm, posv)

        def init(i, carry):
            tokb[pl.ds(i * 16, 16)] = jnp.full((16,), SENT, jnp.int32)
            return carry

        lax.fori_loop(0, SLOTS // 16, init, 0)
        iota16 = lax.broadcasted_iota(jnp.int32, (16,), 0)

        def body(c, carry):
            pv = posv[pl.ds(c * 16, 16)]
            av = c * 16 + iota16
            tv = jnp.bitwise_and(av, S - 1)
            rel = pv - base
            msk = (rel >= 0) & (rel < SLOTS)
            relc = jnp.clip(rel, 0, SLOTS - 1)
            plsc.store_scatter(tokb, [relc], tv, mask=msk)
            return carry

        lax.fori_loop(0, NA // 16, body, 0)
        pltpu.sync_copy(tokb, tok_hbm.at[pl.ds(base, SLOTS)])

    @functools.partial(
        pl.kernel,
        mesh=mesh,
        compiler_params=pltpu.CompilerParams(needs_layout_passes=False),
        out_type=jax.ShapeDtypeStruct((V * TOPK * S, DP), jnp.int32),
        scratch_types=[
            pltpu.VMEM((_GCH,), jnp.int32),
            pltpu.VMEM((_GCH,), jnp.int32),
            pltpu.VMEM((_GCH, DP), jnp.int32),
            pltpu.VMEM((_GCH, DP), jnp.int32),
            pltpu.SemaphoreType.DMA,
            pltpu.SemaphoreType.DMA,
        ],
    )
    def sc_gather_eo(pos_hbm, eo_hbm, g4_hbm, p0, p1, r0, r1, s0, s1):
        pv = (p0, p1)
        bufs = (r0, r1)
        sems = (s0, s1)
        wid = lax.axis_index("s") * _NC + lax.axis_index("c")
        tbase = wid * TT
        nch = TT // _GCH
        total = V * TOPK * nch

        def start(ci):
            j, half = ci // nch, ci % nch
            off = j * S + tbase + half * _GCH
            pltpu.sync_copy(pos_hbm.at[pl.ds(off, _GCH)], pv[ci % 2])
            return pltpu.async_copy(eo_hbm.at[pv[ci % 2]], bufs[ci % 2], sems[ci % 2])

        handles = {0: start(0), 1: start(1)}
        for ci in range(total):
            handles[ci].wait()
            j, half = ci // nch, ci % nch
            off = j * S + tbase + half * _GCH
            pltpu.sync_copy(bufs[ci % 2], g4_hbm.at[pl.ds(off, _GCH)])
            nxt = ci + 2
            if nxt < total:
                handles[nxt] = start(nxt)

    return sc_scatter, sc_gather_eo


def kernel(x, total_logits, total_masks, attn_mask, Wq, bq, Wk, bk, Wv, bv, Wo, bo,
           g1, beta1, g2, beta2, W1v, b1v, W2v, b2v, W1g, b1g, W2g, b2g):
    f32 = jnp.float32
    xf = x.reshape(S, D)

    lg = total_logits.reshape(VS, E)
    mk = total_masks.reshape(VS, E)
    posk, gatek, gbe2, act2, guide2 = pl.pallas_call(
        _route_body,
        in_specs=[
            pl.BlockSpec((VS, E), lambda: (0, 0)),
            pl.BlockSpec((VS, E), lambda: (0, 0)),
        ],
        out_specs=[
            pl.BlockSpec((VS, TOPK), lambda: (0, 0)),
            pl.BlockSpec((VS, TOPK), lambda: (0, 0)),
            pl.BlockSpec((NBT, 1), lambda: (0, 0)),
            pl.BlockSpec((NBT, 1), lambda: (0, 0)),
            pl.BlockSpec((1, 1), lambda: (0, 0)),
        ],
        out_shape=[
            jax.ShapeDtypeStruct((VS, TOPK), jnp.int32),
            jax.ShapeDtypeStruct((VS, TOPK), f32),
            jax.ShapeDtypeStruct((NBT, 1), jnp.int32),
            jax.ShapeDtypeStruct((NBT, 1), jnp.int32),
            jax.ShapeDtypeStruct((1, 1), f32),
        ],
    )(lg, mk)
    total_guide = guide2[0, 0]
    pos4 = posk.reshape(V, S, TOPK).transpose(0, 2, 1).reshape(NA)
    gate4 = gatek.reshape(V, S, TOPK).transpose(1, 0, 2).reshape(S, V * TOPK)
    gbe = gbe2.reshape(NBT)

    sc_scatter, sc_gather_eo = _sc_kernels()
    tokbuf = sc_scatter(pos4)

    Wqkv = jnp.concatenate([Wq, Wk, Wv], axis=1)
    bqkv = jnp.concatenate([bq, bk, bv]).reshape(1, 3 * D)
    qkv = pl.pallas_call(
        _qkv_body,
        grid=(S // BT,),
        in_specs=[
            pl.BlockSpec((BT, D), lambda i: (i, 0)),
            pl.BlockSpec((D, 3 * D), lambda i: (0, 0)),
            pl.BlockSpec((1, 3 * D), lambda i: (0, 0)),
        ],
        out_specs=pl.BlockSpec((BT, 3 * D), lambda i: (i, 0)),
        out_shape=jax.ShapeDtypeStruct((S, 3 * D), jnp.bfloat16),
    )(xf, Wqkv, bqkv)

    QCH = 512
    o_parts = []
    for t in range(S // QCH):
        skv = (t + 1) * QCH
        qoff = t * QCH
        o_parts.append(pl.pallas_call(
            functools.partial(_attn_body, qoff=qoff),
            grid=(H // 2, QCH // BQ),
            in_specs=[
                pl.BlockSpec((BQ, 2 * DH), lambda hh, i, t=t: (t * QCH // BQ + i, hh)),
                pl.BlockSpec((skv, 2 * DH), lambda hh, i: (0, H // 2 + hh)),
                pl.BlockSpec((skv, 2 * DH), lambda hh, i: (0, H + hh)),
            ],
            out_specs=pl.BlockSpec((BQ, 2 * DH), lambda hh, i: (i, hh)),
            out_shape=jax.ShapeDtypeStruct((QCH, D), jnp.bfloat16),
        )(qkv, qkv, qkv))
    o2 = jnp.concatenate(o_parts, axis=0)

    x1 = pl.pallas_call(
        _oproj_ln_body,
        grid=(S // BT,),
        in_specs=[
            pl.BlockSpec((BT, D), lambda i: (i, 0)),
            pl.BlockSpec((D, D), lambda i: (0, 0)),
            pl.BlockSpec((1, D), lambda i: (0, 0)),
            pl.BlockSpec((BT, D), lambda i: (i, 0)),
            pl.BlockSpec((1, D), lambda i: (0, 0)),
            pl.BlockSpec((1, D), lambda i: (0, 0)),
        ],
        out_specs=pl.BlockSpec((BT, D), lambda i: (i, 0)),
        out_shape=jax.ShapeDtypeStruct((S, D), f32),
    )(o2, Wo, bo.reshape(1, D), xf, g1.reshape(1, D), beta1.reshape(1, D))

    tok3 = tokbuf.reshape(NBT, 1, G)
    xb = x1.astype(jnp.bfloat16)
    W1r = W1v.reshape(V * E, D, DFF)
    b1r = b1v.reshape(V * E, 1, DFF)
    W2r = W2v.reshape(V * E, DFF, D)
    b2r = b2v.reshape(V * E, 1, D)
    eo = pl.pallas_call(
        _gmm_body,
        grid_spec=pltpu.PrefetchScalarGridSpec(
            num_scalar_prefetch=2,
            grid=(NBT,),
            in_specs=[
                pl.BlockSpec((1, 1, G), lambda i, gbe_r, act_r: (i, 0, 0)),
                pl.BlockSpec((S, D), lambda i, gbe_r, act_r: (0, 0)),
                pl.BlockSpec((1, D, DFF), lambda i, gbe_r, act_r: (gbe_r[i], 0, 0)),
                pl.BlockSpec((1, 1, DFF), lambda i, gbe_r, act_r: (gbe_r[i], 0, 0)),
                pl.BlockSpec((1, DFF, D), lambda i, gbe_r, act_r: (gbe_r[i], 0, 0)),
                pl.BlockSpec((1, 1, D), lambda i, gbe_r, act_r: (gbe_r[i], 0, 0)),
            ],
            out_specs=pl.BlockSpec((G, DP), lambda i, gbe_r, act_r: (i, 0)),
        ),
        out_shape=jax.ShapeDtypeStruct((2 * P, DP), jnp.int32),
    )(gbe, act2.reshape(NBT), tok3, xb, W1r, b1r, W2r, b2r)

    g4 = sc_gather_eo(pos4, eo).reshape(V * TOPK, S, DP)

    gen = pl.pallas_call(
        _gffn_body,
        grid=(S // BT,),
        in_specs=[
            pl.BlockSpec((BT, D), lambda i: (i, 0)),
            pl.BlockSpec((D, DFFG), lambda i: (0, 0)),
            pl.BlockSpec((1, DFFG), lambda i: (0, 0)),
            pl.BlockSpec((DFFG, D), lambda i: (0, 0)),
            pl.BlockSpec((1, D), lambda i: (0, 0)),
        ],
        out_specs=pl.BlockSpec((BT, D), lambda i: (i, 0)),
        out_shape=jax.ShapeDtypeStruct((S, D), f32),
    )(x1, W1g, b1g.reshape(1, DFFG), W2g, b2g.reshape(1, D))

    out = pl.pallas_call(
        _final_body,
        grid=(S // BT,),
        in_specs=[
            pl.BlockSpec((BT, D), lambda i: (i, 0)),
            pl.BlockSpec((BT, D), lambda i: (i, 0)),
            pl.BlockSpec((V * TOPK, BT, DP), lambda i: (0, i, 0)),
            pl.BlockSpec((BT, V * TOPK), lambda i: (i, 0)),
            pl.BlockSpec((1, D), lambda i: (0, 0)),
            pl.BlockSpec((1, D), lambda i: (0, 0)),
        ],
        out_specs=pl.BlockSpec((BT, D), lambda i: (i, 0)),
        out_shape=jax.ShapeDtypeStruct((S, D), f32),
    )(x1, gen, g4, gate4, g2.reshape(1, D), beta2.reshape(1, D))

    return out.reshape(B, S, D), total_guide


# speedup vs baseline: 2.1112x; 1.0296x over previous
import functools
import math

import jax
import jax.numpy as jnp
from jax import lax
from jax.experimental import pallas as pl
from jax.experimental.pallas import tpu as pltpu
from jax.experimental.pallas import tpu_sc as plsc

B, S, D, H = 1, 2048, 1024, 16
V, E, TOPK = 2, 8, 2
DFF, DFFG = 1024, 2048
DH = D // H
DP = D // 2

BT = 512
BQ = 512
G = 256
P = 6144
NB = P // G
NBT = V * NB
VS = V * S
NA = V * S * TOPK

_NC, _NS = 2, 16
NW = _NC * _NS
SLOTS = 2 * P // NW
TT = S // NW
SENT = 0


def _qkv_body(x_ref, w_ref, b_ref, o_ref):
    acc = jnp.dot(
        x_ref[...].astype(jnp.bfloat16),
        w_ref[...].astype(jnp.bfloat16),
        preferred_element_type=jnp.float32,
    )
    o_ref[...] = (acc + b_ref[...]).astype(jnp.bfloat16)


def _attn_body(q_ref, k_ref, v_ref, o_ref, *, qoff):
    qi = pl.program_id(1)
    skv = k_ref.shape[0]
    for h2 in range(2):
        cs = h2 * DH
        q = q_ref[:, cs:cs + DH]
        k = k_ref[:, cs:cs + DH]
        v = v_ref[:, cs:cs + DH]
        sc = lax.dot_general(
            q, k, (((1,), (1,)), ((), ())), preferred_element_type=jnp.float32
        ) / math.sqrt(DH)
        w = skv - qoff
        rows = qi * BQ + lax.broadcasted_iota(jnp.int32, (BQ, w), 0)
        cols = lax.broadcasted_iota(jnp.int32, (BQ, w), 1)
        tail = jnp.where(cols > rows, jnp.float32(-1e9), sc[:, qoff:])
        if qoff:
            sc = jnp.concatenate([sc[:, :qoff], tail], axis=1)
        else:
            sc = tail
        pr = jax.nn.softmax(sc, axis=-1)
        o_ref[:, cs:cs + DH] = jnp.dot(
            pr.astype(jnp.bfloat16), v, preferred_element_type=jnp.float32
        ).astype(jnp.bfloat16)


def _oproj_ln_body(o_ref, w_ref, b_ref, x_ref, g_ref, beta_ref, out_ref):
    y = (
        jnp.dot(
            o_ref[...],
            w_ref[...].astype(jnp.bfloat16),
            preferred_element_type=jnp.float32,
        )
        + b_ref[...]
        + x_ref[...]
    )
    m = jnp.mean(y, axis=-1, keepdims=True)
    v = jnp.mean((y - m) ** 2, axis=-1, keepdims=True)
    out_ref[...] = (y - m) * lax.rsqrt(v + 1e-5) * g_ref[...] + beta_ref[...]


def _cumsum_rows(x):
    n = x.shape[0]
    s = 1
    while s < n:
        x = x + jnp.concatenate(
            [jnp.zeros((s, x.shape[1]), x.dtype), x[:-s, :]], axis=0
        )
        s *= 2
    return x


def _route_body(lg_ref, mk_ref, posk_ref, gatek_ref, gbe_ref, act_ref, guide_ref):
    lg = lg_ref[...]
    probs = jax.nn.softmax(lg, axis=-1)
    iota_e = lax.broadcasted_iota(jnp.int32, (VS, E), 1)
    m1 = jnp.max(probs, axis=-1, keepdims=True)
    i1 = jnp.min(jnp.where(probs == m1, iota_e, E), axis=-1, keepdims=True)
    oh1 = iota_e == i1
    p2 = jnp.where(oh1, jnp.float32(-1.0), probs)
    m2 = jnp.max(p2, axis=-1, keepdims=True)
    i2 = jnp.min(jnp.where(p2 == m2, iota_e, E), axis=-1, keepdims=True)
    oh2 = iota_e == i2
    ssum = m1 + m2
    gatek_ref[...] = jnp.concatenate([m1 / ssum, m2 / ssum], axis=1)

    mk = mk_ref[...]
    mn = mk / (jnp.sum(mk, axis=-1, keepdims=True) + 1e-9)
    guide_ref[...] = (-jnp.sum(mn * jnp.log(probs + 1e-9)) / (S * V)).reshape(1, 1)

    cnt = (oh1 | oh2).astype(jnp.int32)
    C = _cumsum_rows(cnt)
    n0 = C[S - 1:S, :]
    n1 = C[VS - 1:VS, :] - n0
    rows = lax.broadcasted_iota(jnp.int32, (VS, 1), 0)
    is_v1 = rows >= S
    excl = C - cnt - jnp.where(is_v1, 1, 0) * n0
    np0 = ((n0 + (G - 1)) // G) * G
    np1 = ((n1 + (G - 1)) // G) * G
    tri = (
        lax.broadcasted_iota(jnp.int32, (E, E), 0)
        < lax.broadcasted_iota(jnp.int32, (E, E), 1)
    ).astype(jnp.float32)
    po0 = jnp.dot(
        np0.astype(jnp.float32), tri, preferred_element_type=jnp.float32
    ).astype(jnp.int32)
    po1 = jnp.dot(
        np1.astype(jnp.float32), tri, preferred_element_type=jnp.float32
    ).astype(jnp.int32)
    po_full = jnp.where(is_v1, po1, po0)
    base = po_full + excl + jnp.where(is_v1, P, 0)
    pos0 = jnp.sum(jnp.where(oh1, base, 0), axis=-1, keepdims=True)
    pos1 = jnp.sum(jnp.where(oh2, base, 0), axis=-1, keepdims=True)
    posk_ref[...] = jnp.concatenate([pos0, pos1], axis=1)

    r = lax.broadcasted_iota(jnp.int32, (NBT, 1), 0)
    isb1 = r >= NB
    iloc = jnp.where(isb1, r - NB, r)
    ends = jnp.where(isb1, po1 + np1, po0 + np0)
    cntb = jnp.sum((ends <= iloc * G).astype(jnp.int32), axis=-1, keepdims=True)
    gbe_ref[...] = jnp.minimum(cntb, E - 1) + jnp.where(isb1, E, 0)
    act_ref[...] = (iloc * G < ends[:, E - 1:E]).astype(jnp.int32)


def _gmm_body(gbe_ref, act_ref, tok_ref, xb_ref, w1_ref, b1_ref, w2_ref, b2_ref,
              eo_ref):
    @pl.when(act_ref[pl.program_id(0)] == 1)
    def _():
        _gmm_inner(tok_ref, xb_ref, w1_ref, b1_ref, w2_ref, b2_ref, eo_ref)


def _gmm_inner(tok_ref, xb_ref, w1_ref, b1_ref, w2_ref, b2_ref, eo_ref):
    tids = tok_ref[0, 0, :]
    onehot = (
        tids[:, None] == lax.broadcasted_iota(jnp.int32, (G, S), 1)
    ).astype(jnp.bfloat16)
    xg = jnp.dot(onehot, xb_ref[...], preferred_element_type=jnp.float32)
    h = jax.nn.gelu(
        jnp.dot(
            xg.astype(jnp.bfloat16),
            w1_ref[0, :, :].astype(jnp.bfloat16),
            preferred_element_type=jnp.float32,
        )
        + b1_ref[0, :, :]
    )
    eo = (
        jnp.dot(
            h.astype(jnp.bfloat16),
            w2_ref[0, :, :].astype(jnp.bfloat16),
            preferred_element_type=jnp.float32,
        )
        + b2_ref[0, :, :]
    )
    ai = lax.bitcast_convert_type(eo[:, :DP], jnp.int32)
    bi = lax.bitcast_convert_type(eo[:, DP:], jnp.int32)
    eo_ref[...] = lax.shift_right_logical(ai, 16) | (bi & jnp.int32(-65536))


def _final_body(x1_ref, w1_ref, b1_ref, w2_ref, b2_ref, g4_ref, gk_ref, g_ref,
                beta_ref, out_ref):
    x = x1_ref[...]
    hgen = jax.nn.gelu(
        jnp.dot(
            x.astype(jnp.bfloat16),
            w1_ref[...].astype(jnp.bfloat16),
            preferred_element_type=jnp.float32,
        )
        + b1_ref[...]
    )
    gen = (
        jnp.dot(
            hgen.astype(jnp.bfloat16),
            w2_ref[...].astype(jnp.bfloat16),
            preferred_element_type=jnp.float32,
        )
        + b2_ref[...]
    )
    gk = gk_ref[...]
    lo = gen[:, :DP] + x[:, :DP]
    hi = gen[:, DP:] + x[:, DP:]
    for j in range(V * TOPK):
        w = g4_ref[j, :, :]
        a = lax.bitcast_convert_type(lax.shift_left(w, 16), jnp.float32)
        b = lax.bitcast_convert_type(w & jnp.int32(-65536), jnp.float32)
        gj = gk[:, j:j + 1]
        lo = lo + a * gj
        hi = hi + b * gj
    fin = jnp.concatenate([lo, hi], axis=1)
    m = jnp.mean(fin, axis=-1, keepdims=True)
    v = jnp.mean((fin - m) ** 2, axis=-1, keepdims=True)
    out_ref[...] = (fin - m) * lax.rsqrt(v + 1e-5) * g_ref[...] + beta_ref[...]


_GCH = 32


@functools.lru_cache(maxsize=None)
def _sc_kernels():
    mesh = plsc.VectorSubcoreMesh(core_axis_name="c", subcore_axis_name="s")

    @functools.partial(
        pl.kernel,
        mesh=mesh,
        compiler_params=pltpu.CompilerParams(needs_layout_passes=False),
        out_type=jax.ShapeDtypeStruct((2 * P,), jnp.int32),
        scratch_types=[
            pltpu.VMEM((NA,), jnp.int32),
            pltpu.VMEM((SLOTS,), jnp.int32),
        ],
    )
    def sc_scatter(pos_hbm, tok_hbm, posv, tokb):
        wid = lax.axis_index("s") * _NC + lax.axis_index("c")
        base = wid * SLOTS
        pltpu.sync_copy(pos_hbm, posv)

        def init(i, carry):
            tokb[pl.ds(i * 16, 16)] = jnp.full((16,), SENT, jnp.int32)
            return carry

        lax.fori_loop(0, SLOTS // 16, init, 0)
        iota16 = lax.broadcasted_iota(jnp.int32, (16,), 0)

        def body(c, carry):
            pv = posv[pl.ds(c * 16, 16)]
            av = c * 16 + iota16
            tv = jnp.bitwise_and(av, S - 1)
            rel = pv - base
            msk = (rel >= 0) & (rel < SLOTS)
            relc = jnp.clip(rel, 0, SLOTS - 1)
            plsc.store_scatter(tokb, [relc], tv, mask=msk)
            return carry

        lax.fori_loop(0, NA // 16, body, 0)
        pltpu.sync_copy(tokb, tok_hbm.at[pl.ds(base, SLOTS)])

    @functools.partial(
        pl.kernel,
        mesh=mesh,
        compiler_params=pltpu.CompilerParams(needs_layout_passes=False),
        out_type=jax.ShapeDtypeStruct((V * TOPK * S, DP), jnp.int32),
        scratch_types=[
            pltpu.VMEM((_GCH,), jnp.int32),
            pltpu.VMEM((_GCH,), jnp.int32),
            pltpu.VMEM((_GCH, DP), jnp.int32),
            pltpu.VMEM((_GCH, DP), jnp.int32),
            pltpu.SemaphoreType.DMA,
            pltpu.SemaphoreType.DMA,
        ],
    )
    def sc_gather_eo(pos_hbm, eo_hbm, g4_hbm, p0, p1, r0, r1, s0, s1):
        pv = (p0, p1)
        bufs = (r0, r1)
        sems = (s0, s1)
        wid = lax.axis_index("s") * _NC + lax.axis_index("c")
        tbase = wid * TT
        nch = TT // _GCH
        total = V * TOPK * nch

        def start(ci):
            j, half = ci // nch, ci % nch
            off = j * S + tbase + half * _GCH
            pltpu.sync_copy(pos_hbm.at[pl.ds(off, _GCH)], pv[ci % 2])
            return pltpu.async_copy(eo_hbm.at[pv[ci % 2]], bufs[ci % 2], sems[ci % 2])

        handles = {0: start(0), 1: start(1)}
        for ci in range(total):
            handles[ci].wait()
            j, half = ci // nch, ci % nch
            off = j * S + tbase + half * _GCH
            pltpu.sync_copy(bufs[ci % 2], g4_hbm.at[pl.ds(off, _GCH)])
            nxt = ci + 2
            if nxt < total:
                handles[nxt] = start(nxt)

    return sc_scatter, sc_gather_eo


def kernel(x, total_logits, total_masks, attn_mask, Wq, bq, Wk, bk, Wv, bv, Wo, bo,
           g1, beta1, g2, beta2, W1v, b1v, W2v, b2v, W1g, b1g, W2g, b2g):
    f32 = jnp.float32
    xf = x.reshape(S, D)

    lg = total_logits.reshape(VS, E)
    mk = total_masks.reshape(VS, E)
    posk, gatek, gbe2, act2, guide2 = pl.pallas_call(
        _route_body,
        in_specs=[
            pl.BlockSpec((VS, E), lambda: (0, 0)),
            pl.BlockSpec((VS, E), lambda: (0, 0)),
        ],
        out_specs=[
            pl.BlockSpec((VS, TOPK), lambda: (0, 0)),
            pl.BlockSpec((VS, TOPK), lambda: (0, 0)),
            pl.BlockSpec((NBT, 1), lambda: (0, 0)),
            pl.BlockSpec((NBT, 1), lambda: (0, 0)),
            pl.BlockSpec((1, 1), lambda: (0, 0)),
        ],
        out_shape=[
            jax.ShapeDtypeStruct((VS, TOPK), jnp.int32),
            jax.ShapeDtypeStruct((VS, TOPK), f32),
            jax.ShapeDtypeStruct((NBT, 1), jnp.int32),
            jax.ShapeDtypeStruct((NBT, 1), jnp.int32),
            jax.ShapeDtypeStruct((1, 1), f32),
        ],
    )(lg, mk)
    total_guide = guide2[0, 0]
    pos4 = posk.reshape(V, S, TOPK).transpose(0, 2, 1).reshape(NA)
    gate4 = gatek.reshape(V, S, TOPK).transpose(1, 0, 2).reshape(S, V * TOPK)
    gbe = gbe2.reshape(NBT)

    sc_scatter, sc_gather_eo = _sc_kernels()
    tokbuf = sc_scatter(pos4)

    Wqkv = jnp.concatenate([Wq, Wk, Wv], axis=1)
    bqkv = jnp.concatenate([bq, bk, bv]).reshape(1, 3 * D)
    qkv = pl.pallas_call(
        _qkv_body,
        grid=(S // BT,),
        in_specs=[
            pl.BlockSpec((BT, D), lambda i: (i, 0)),
            pl.BlockSpec((D, 3 * D), lambda i: (0, 0)),
            pl.BlockSpec((1, 3 * D), lambda i: (0, 0)),
        ],
        out_specs=pl.BlockSpec((BT, 3 * D), lambda i: (i, 0)),
        out_shape=jax.ShapeDtypeStruct((S, 3 * D), jnp.bfloat16),
    )(xf, Wqkv, bqkv)

    QCH = 512
    o_parts = []
    for t in range(S // QCH):
        skv = (t + 1) * QCH
        qoff = t * QCH
        o_parts.append(pl.pallas_call(
            functools.partial(_attn_body, qoff=qoff),
            grid=(H // 2, QCH // BQ),
            in_specs=[
                pl.BlockSpec((BQ, 2 * DH), lambda hh, i, t=t: (t * QCH // BQ + i, hh)),
                pl.BlockSpec((skv, 2 * DH), lambda hh, i: (0, H // 2 + hh)),
                pl.BlockSpec((skv, 2 * DH), lambda hh, i: (0, H + hh)),
            ],
            out_specs=pl.BlockSpec((BQ, 2 * DH), lambda hh, i: (i, hh)),
            out_shape=jax.ShapeDtypeStruct((QCH, D), jnp.bfloat16),
        )(qkv, qkv, qkv))
    o2 = jnp.concatenate(o_parts, axis=0)

    x1 = pl.pallas_call(
        _oproj_ln_body,
        grid=(S // BT,),
        in_specs=[
            pl.BlockSpec((BT, D), lambda i: (i, 0)),
            pl.BlockSpec((D, D), lambda i: (0, 0)),
            pl.BlockSpec((1, D), lambda i: (0, 0)),
            pl.BlockSpec((BT, D), lambda i: (i, 0)),
            pl.BlockSpec((1, D), lambda i: (0, 0)),
            pl.BlockSpec((1, D), lambda i: (0, 0)),
        ],
        out_specs=pl.BlockSpec((BT, D), lambda i: (i, 0)),
        out_shape=jax.ShapeDtypeStruct((S, D), f32),
    )(o2, Wo, bo.reshape(1, D), xf, g1.reshape(1, D), beta1.reshape(1, D))

    tok3 = tokbuf.reshape(NBT, 1, G)
    xb = x1.astype(jnp.bfloat16)
    W1r = W1v.reshape(V * E, D, DFF)
    b1r = b1v.reshape(V * E, 1, DFF)
    W2r = W2v.reshape(V * E, DFF, D)
    b2r = b2v.reshape(V * E, 1, D)
    eo = pl.pallas_call(
        _gmm_body,
        grid_spec=pltpu.PrefetchScalarGridSpec(
            num_scalar_prefetch=2,
            grid=(NBT,),
            in_specs=[
                pl.BlockSpec((1, 1, G), lambda i, gbe_r, act_r: (i, 0, 0)),
                pl.BlockSpec((S, D), lambda i, gbe_r, act_r: (0, 0)),
                pl.BlockSpec((1, D, DFF), lambda i, gbe_r, act_r: (gbe_r[i], 0, 0)),
                pl.BlockSpec((1, 1, DFF), lambda i, gbe_r, act_r: (gbe_r[i], 0, 0)),
                pl.BlockSpec((1, DFF, D), lambda i, gbe_r, act_r: (gbe_r[i], 0, 0)),
                pl.BlockSpec((1, 1, D), lambda i, gbe_r, act_r: (gbe_r[i], 0, 0)),
            ],
            out_specs=pl.BlockSpec((G, DP), lambda i, gbe_r, act_r: (i, 0)),
        ),
        out_shape=jax.ShapeDtypeStruct((2 * P, DP), jnp.int32),
    )(gbe, act2.reshape(NBT), tok3, xb, W1r, b1r, W2r, b2r)

    g4 = sc_gather_eo(pos4, eo).reshape(V * TOPK, S, DP)

    out = pl.pallas_call(
        _final_body,
        grid=(S // BT,),
        in_specs=[
            pl.BlockSpec((BT, D), lambda i: (i, 0)),
            pl.BlockSpec((D, DFFG), lambda i: (0, 0)),
            pl.BlockSpec((1, DFFG), lambda i: (0, 0)),
            pl.BlockSpec((DFFG, D), lambda i: (0, 0)),
            pl.BlockSpec((1, D), lambda i: (0, 0)),
            pl.BlockSpec((V * TOPK, BT, DP), lambda i: (0, i, 0)),
            pl.BlockSpec((BT, V * TOPK), lambda i: (i, 0)),
            pl.BlockSpec((1, D), lambda i: (0, 0)),
            pl.BlockSpec((1, D), lambda i: (0, 0)),
        ],
        out_specs=pl.BlockSpec((BT, D), lambda i: (i, 0)),
        out_shape=jax.ShapeDtypeStruct((S, D), f32),
    )(x1, W1g, b1g.reshape(1, DFFG), W2g, b2g.reshape(1, D), g4, gate4,
      g2.reshape(1, D), beta2.reshape(1, D))

    return out.reshape(B, S, D), total_guide
